# speedup vs baseline: 1.5027x; 1.5027x over previous
import functools

import jax
import jax.numpy as jnp
from jax import lax
from jax.experimental import pallas as pl
from jax.experimental.pallas import tpu as pltpu
from jax.experimental.pallas import tpu_sc as plsc

LANES = 16


def _build_sc_embed(n, emb, chunk, n_chunks, rows_per_w, nc):
    mesh = plsc.VectorSubcoreMesh(core_axis_name="c", subcore_axis_name="s")

    @functools.partial(
        pl.kernel,
        out_type=(
            jax.ShapeDtypeStruct((n, emb), jnp.float32),
            jax.ShapeDtypeStruct((n, emb), jnp.float32),
        ),
        mesh=mesh,
        scratch_types=[
            pltpu.VMEM((chunk,), jnp.int32),
            pltpu.VMEM((chunk,), jnp.int32),
            pltpu.VMEM((chunk, emb), jnp.float32),
            pltpu.VMEM((chunk, emb), jnp.float32),
            pltpu.SemaphoreType.DMA,
            pltpu.SemaphoreType.DMA,
        ],
    )
    def body(tok_hbm, pos_hbm, emb_hbm, cache_hbm, x_hbm, pe_hbm,
             tok_idx, pos_idx, tok_buf, pos_buf, sem_t, sem_p):
        wid = lax.axis_index("s") * nc + lax.axis_index("c")
        wbase = wid * rows_per_w

        def chunk_body(c, carry):
            base = wbase + c * chunk
            pltpu.sync_copy(tok_hbm.at[pl.ds(base, chunk)], tok_idx)
            pltpu.sync_copy(pos_hbm.at[pl.ds(base, chunk)], pos_idx)
            cp_t = pltpu.async_copy(emb_hbm.at[tok_idx], tok_buf, sem_t)
            cp_p = pltpu.async_copy(cache_hbm.at[pos_idx], pos_buf, sem_p)
            cp_t.wait()
            cp_p.wait()
            pltpu.sync_copy(pos_buf, pe_hbm.at[pl.ds(base, chunk)])

            def row_body(r, rc):
                def col_body(i, cc):
                    sl = pl.ds(i * LANES, LANES)
                    plsc.addupdate(tok_buf.at[r, sl], pos_buf[r, sl])
                    return cc
                return lax.fori_loop(0, emb // LANES, col_body, rc)

            lax.fori_loop(0, chunk, row_body, 0)
            pltpu.sync_copy(tok_buf, x_hbm.at[pl.ds(base, chunk)])
            return carry

        lax.fori_loop(0, n_chunks, chunk_body, 0)

    return body


def kernel(tokens, mask, pos_ids, embedding, pos_emb_cache):
    bs, seq = tokens.shape
    _, emb = embedding.shape
    n = bs * seq

    info = plsc.get_sparse_core_info()
    nc, ns = info.num_cores, info.num_subcores
    nw = nc * ns
    rows_per_w = n // nw
    chunk = 16
    n_chunks = rows_per_w // chunk

    tok_flat = tokens.reshape(n).astype(jnp.int32)
    pos_flat = pos_ids.reshape(n).astype(jnp.int32)

    body = _build_sc_embed(n, emb, chunk, n_chunks, rows_per_w, nc)
    x_flat, pe_flat = body(tok_flat, pos_flat, embedding, pos_emb_cache)
    x = x_flat.reshape(bs, seq, emb)
    pe = pe_flat.reshape(bs, seq, emb)
    return (x, mask, pe)


# speedup vs baseline: 3.4848x; 2.3190x over previous
import functools

import jax
import jax.numpy as jnp
from jax import lax
from jax.experimental import pallas as pl
from jax.experimental.pallas import tpu as pltpu
from jax.experimental.pallas import tpu_sc as plsc

LANES = 16
NBUF = 2


def _build_sc_embed(n, emb, chunk, n_chunks, rows_per_w, nc):
    mesh = plsc.VectorSubcoreMesh(core_axis_name="c", subcore_axis_name="s")
    n_groups = n_chunks // NBUF

    scratch = []
    for _ in range(NBUF):
        scratch += [
            pltpu.VMEM((chunk,), jnp.int32),
            pltpu.VMEM((chunk,), jnp.int32),
            pltpu.VMEM((chunk, emb), jnp.float32),
            pltpu.VMEM((chunk, emb), jnp.float32),
        ]
    scratch += [pltpu.SemaphoreType.DMA] * (4 * NBUF)

    @functools.partial(
        pl.kernel,
        out_type=(
            jax.ShapeDtypeStruct((n, emb), jnp.float32),
            jax.ShapeDtypeStruct((n, emb), jnp.float32),
        ),
        mesh=mesh,
        scratch_types=scratch,
    )
    def body(tok_hbm, pos_hbm, emb_hbm, cache_hbm, x_hbm, pe_hbm, *s):
        bufs = [s[4 * b:4 * b + 4] for b in range(NBUF)]
        sems = [s[4 * NBUF + 4 * b:4 * NBUF + 4 * b + 4] for b in range(NBUF)]

        wid = lax.axis_index("s") * nc + lax.axis_index("c")
        wbase = wid * rows_per_w

        def issue(c, b):
            ti, pi, tb, pb = bufs[b]
            sem_gt, sem_gp, _, _ = sems[b]
            base = wbase + c * chunk
            pltpu.sync_copy(tok_hbm.at[pl.ds(base, chunk)], ti)
            pltpu.sync_copy(pos_hbm.at[pl.ds(base, chunk)], pi)
            pltpu.async_copy(emb_hbm.at[ti], tb, sem_gt)
            pltpu.async_copy(cache_hbm.at[pi], pb, sem_gp)

        def add_rows(tb, pb):
            def row_body(r, carry):
                for i in range(emb // LANES):
                    sl = pl.ds(i * LANES, LANES)
                    plsc.addupdate(tb.at[r, sl], pb[r, sl])
                return carry
            lax.fori_loop(0, chunk, row_body, 0)

        for b in range(NBUF):
            issue(b, b)

        def group_body(g, carry):
            for b in range(NBUF):
                c = g * NBUF + b
                ti, pi, tb, pb = bufs[b]
                sem_gt, sem_gp, sem_wx, sem_wp = sems[b]
                base = wbase + c * chunk
                pltpu.make_async_copy(emb_hbm.at[ti], tb, sem_gt).wait()
                pltpu.make_async_copy(cache_hbm.at[pi], pb, sem_gp).wait()
                pltpu.async_copy(pb, pe_hbm.at[pl.ds(base, chunk)], sem_wp)
                add_rows(tb, pb)
                pltpu.async_copy(tb, x_hbm.at[pl.ds(base, chunk)], sem_wx)

                nxt = c + NBUF

                @pl.when(nxt < n_chunks)
                def _prefetch():
                    pltpu.make_async_copy(
                        tb, x_hbm.at[pl.ds(0, chunk)], sem_wx).wait()
                    pltpu.make_async_copy(
                        pb, pe_hbm.at[pl.ds(0, chunk)], sem_wp).wait()
                    issue(nxt, b)
            return carry

        lax.fori_loop(0, n_groups, group_body, 0)

        for b in range(NBUF):
            _, _, tb, pb = bufs[b]
            _, _, sem_wx, sem_wp = sems[b]
            pltpu.make_async_copy(tb, x_hbm.at[pl.ds(0, chunk)], sem_wx).wait()
            pltpu.make_async_copy(pb, pe_hbm.at[pl.ds(0, chunk)], sem_wp).wait()

    return body


def kernel(tokens, mask, pos_ids, embedding, pos_emb_cache):
    bs, seq = tokens.shape
    _, emb = embedding.shape
    n = bs * seq

    info = plsc.get_sparse_core_info()
    nc, ns = info.num_cores, info.num_subcores
    nw = nc * ns
    rows_per_w = n // nw
    chunk = 16
    n_chunks = rows_per_w // chunk

    tok_flat = tokens.reshape(n).astype(jnp.int32)
    pos_flat = pos_ids.reshape(n).astype(jnp.int32)

    body = _build_sc_embed(n, emb, chunk, n_chunks, rows_per_w, nc)
    x_flat, pe_flat = body(tok_flat, pos_flat, embedding, pos_emb_cache)
    x = x_flat.reshape(bs, seq, emb)
    pe = pe_flat.reshape(bs, seq, emb)
    return (x, mask, pe)
